# baseline (speedup 1.0000x reference)
import functools

import numpy as np

import jax
import jax.numpy as jnp
from jax import lax
from jax.experimental import pallas as pl
from jax.experimental.pallas import tpu as pltpu
from jax.experimental.pallas import tpu_sc as plsc

B, N = 4096, 1000
NC, NS, L = 2, 16, 16
NW = NC * NS
ROWS_PER_W = B // NW
HALF = 64
K = 4096
CH = K // L

_mesh = plsc.VectorSubcoreMesh(core_axis_name="c", subcore_axis_name="s")


RP2 = ROWS_PER_W // 2
_HI_MASK = np.int32(-65536)


QROWS = 32
NQ = ROWS_PER_W // QROWS


@functools.partial(
    pl.kernel,
    out_type=jax.ShapeDtypeStruct((NW, L), jnp.float32),
    mesh=_mesh,
    scratch_types=[
        pltpu.VMEM((N * RP2,), jnp.int32),
        pltpu.VMEM((QROWS, N), jnp.float32),
        pltpu.VMEM((K,), jnp.int32),
        pltpu.VMEM((K,), jnp.int32),
        pltpu.VMEM((L,), jnp.float32),
    ],
    compiler_params=pltpu.CompilerParams(
        use_tc_tiling_on_sc=True, needs_layout_passes=False),
)
def _sc_implies(p_hbm, ia_hbm, ib_hbm, out_hbm, slab, fq, ia, ib, obuf):
    wid = lax.axis_index("s") * NC + lax.axis_index("c")
    pltpu.sync_copy(ia_hbm, ia)
    pltpu.sync_copy(ib_hbm, ib)

    def pack16(re, o0):
        def col_body(j, _):
            c = j * L
            a = fq[re, pl.ds(c, L)]
            b = fq[re + 1, pl.ds(c, L)]
            w = plsc.pack(a, b, format=plsc.PackFormat.INTERLEAVED)
            slab[pl.ds(o0 + c, L)] = plsc.bitcast(w, jnp.int32)
            return 0
        lax.fori_loop(0, N // L, col_body, 0, unroll=8)
        c = N - L
        a = fq[re, pl.ds(c, L)]
        b = fq[re + 1, pl.ds(c, L)]
        w = plsc.pack(a, b, format=plsc.PackFormat.INTERLEAVED)
        slab[pl.ds(o0 + c, L)] = plsc.bitcast(w, jnp.int32)

    for q in range(NQ):
        row0 = wid * ROWS_PER_W + q * QROWS
        pltpu.sync_copy(p_hbm.at[pl.ds(row0, QROWS), :], fq)

        def rp_body(rp, _):
            pack16(2 * rp, (q * (QROWS // 2) + rp) * N)
            return 0
        lax.fori_loop(0, QROWS // 2, rp_body, 0)

    def chunk_body(c, tot):
        av = ia[pl.ds(c * L, L)]
        bv = ib[pl.ds(c * L, L)]

        def group_body(g, carry):
            ja, jb, t = carry
            acc = jnp.zeros((2 * L,), jnp.bfloat16)
            for _ in range(4):
                wa = plsc.load_gather(slab, [ja])
                wb = plsc.load_gather(slab, [jb])
                da = plsc.bitcast(wa, jnp.bfloat16)
                db = plsc.bitcast(wb, jnp.bfloat16)
                acc = acc + jnp.maximum(da - db, 0)
                ja = ja + N
                jb = jb + N
            ai = plsc.bitcast(acc, jnp.int32)
            t = t + plsc.bitcast(lax.shift_left(ai, 16), jnp.float32)
            t = t + plsc.bitcast(lax.bitwise_and(ai, _HI_MASK), jnp.float32)
            return ja, jb, t

        _, _, tot = lax.fori_loop(0, RP2 // 4, group_body, (av, bv, tot),
                                  unroll=4)
        return tot

    total = lax.fori_loop(0, CH, chunk_body, jnp.zeros((L,), jnp.float32))
    obuf[...] = total
    pltpu.sync_copy(obuf, out_hbm.at[wid])


BLK = 512
NBLK = B // BLK


def _tc_mutex_body(p_ref, ma_ref, mb_ref, out_ref, g_acc, m_acc):
    i = pl.program_id(0)
    pb = p_ref[...]
    g_part = lax.dot_general(pb, pb, (((0,), (0,)), ((), ())),
                             preferred_element_type=jnp.float32)
    am = ma_ref[0, 0, :]
    bm = mb_ref[0, 0, :]
    cols = lax.broadcasted_iota(jnp.int32, (BLK, N), 1)
    a_oh = (cols == am[:, None]).astype(jnp.bfloat16)
    b_oh = (cols == bm[:, None]).astype(jnp.bfloat16)
    m_part = lax.dot_general(a_oh, b_oh, (((0,), (0,)), ((), ())),
                             preferred_element_type=jnp.float32)

    @pl.when(i == 0)
    def _():
        g_acc[...] = g_part
        m_acc[...] = m_part

    @pl.when(i > 0)
    def _():
        g_acc[...] += g_part
        m_acc[...] += m_part

    @pl.when(i == NBLK - 1)
    def _():
        out_ref[...] = jnp.sum(g_acc[...] * m_acc[...]).reshape(1, 1)


_tc_mutex = pl.pallas_call(
    _tc_mutex_body,
    grid=(NBLK,),
    in_specs=[
        pl.BlockSpec((BLK, N), lambda i: (i, 0)),
        pl.BlockSpec((1, 1, BLK), lambda i: (i, 0, 0)),
        pl.BlockSpec((1, 1, BLK), lambda i: (i, 0, 0)),
    ],
    out_specs=pl.BlockSpec((1, 1), lambda i: (0, 0)),
    out_shape=jax.ShapeDtypeStruct((1, 1), jnp.float32),
    scratch_shapes=[
        pltpu.VMEM((N, N), jnp.float32),
        pltpu.VMEM((N, N), jnp.float32),
    ],
)


def kernel(predictions, mutex_pairs, implies_pairs):
    ma = mutex_pairs[:, 0].astype(jnp.int32).reshape(NBLK, 1, BLK)
    mb = mutex_pairs[:, 1].astype(jnp.int32).reshape(NBLK, 1, BLK)
    ia = implies_pairs[:, 0].astype(jnp.int32)
    ib = implies_pairs[:, 1].astype(jnp.int32)
    pbf = predictions.astype(jnp.bfloat16)
    partials = _sc_implies(predictions, ia, ib)
    mutex_sum = _tc_mutex(pbf, ma, mb)[0, 0]
    return (jnp.sum(partials) + mutex_sum) * (1.0 / B)


# speedup vs baseline: 1.0776x; 1.0776x over previous
import functools

import numpy as np

import jax
import jax.numpy as jnp
from jax import lax
from jax.experimental import pallas as pl
from jax.experimental.pallas import tpu as pltpu
from jax.experimental.pallas import tpu_sc as plsc

B, N = 4096, 1000
NC, NS, L = 2, 16, 16
NW = NC * NS
ROWS_PER_W = B // NW
HALF = 64
K = 4096
K_TC = 1024
K_SC = K - K_TC
CH = K_SC // L

_mesh = plsc.VectorSubcoreMesh(core_axis_name="c", subcore_axis_name="s")


RP2 = ROWS_PER_W // 2
_HI_MASK = np.int32(-65536)


QROWS = 32
NQ = ROWS_PER_W // QROWS


@functools.partial(
    pl.kernel,
    out_type=jax.ShapeDtypeStruct((NW, L), jnp.float32),
    mesh=_mesh,
    scratch_types=[
        pltpu.VMEM((N * RP2,), jnp.int32),
        pltpu.VMEM((QROWS, N), jnp.float32),
        pltpu.VMEM((K_SC,), jnp.int32),
        pltpu.VMEM((K_SC,), jnp.int32),
        pltpu.VMEM((L,), jnp.float32),
    ],
    compiler_params=pltpu.CompilerParams(
        use_tc_tiling_on_sc=True, needs_layout_passes=False),
)
def _sc_implies(p_hbm, ia_hbm, ib_hbm, out_hbm, slab, fq, ia, ib, obuf):
    wid = lax.axis_index("s") * NC + lax.axis_index("c")
    pltpu.sync_copy(ia_hbm, ia)
    pltpu.sync_copy(ib_hbm, ib)

    def pack16(re, o0):
        def col_body(j, _):
            c = j * L
            a = fq[re, pl.ds(c, L)]
            b = fq[re + 1, pl.ds(c, L)]
            w = plsc.pack(a, b, format=plsc.PackFormat.INTERLEAVED)
            slab[pl.ds(o0 + c, L)] = plsc.bitcast(w, jnp.int32)
            return 0
        lax.fori_loop(0, N // L, col_body, 0, unroll=8)
        c = N - L
        a = fq[re, pl.ds(c, L)]
        b = fq[re + 1, pl.ds(c, L)]
        w = plsc.pack(a, b, format=plsc.PackFormat.INTERLEAVED)
        slab[pl.ds(o0 + c, L)] = plsc.bitcast(w, jnp.int32)

    for q in range(NQ):
        row0 = wid * ROWS_PER_W + q * QROWS
        pltpu.sync_copy(p_hbm.at[pl.ds(row0, QROWS), :], fq)

        def rp_body(rp, _):
            pack16(2 * rp, (q * (QROWS // 2) + rp) * N)
            return 0
        lax.fori_loop(0, QROWS // 2, rp_body, 0)

    def chunk_body(c, tot):
        av = ia[pl.ds(c * L, L)]
        bv = ib[pl.ds(c * L, L)]

        def group_body(g, carry):
            ja, jb, t = carry
            acc = jnp.zeros((2 * L,), jnp.bfloat16)
            for _ in range(4):
                wa = plsc.load_gather(slab, [ja])
                wb = plsc.load_gather(slab, [jb])
                da = plsc.bitcast(wa, jnp.bfloat16)
                db = plsc.bitcast(wb, jnp.bfloat16)
                acc = acc + jnp.maximum(da - db, 0)
                ja = ja + N
                jb = jb + N
            ai = plsc.bitcast(acc, jnp.int32)
            t = t + plsc.bitcast(lax.shift_left(ai, 16), jnp.float32)
            t = t + plsc.bitcast(lax.bitwise_and(ai, _HI_MASK), jnp.float32)
            return ja, jb, t

        _, _, tot = lax.fori_loop(0, RP2 // 4, group_body, (av, bv, tot),
                                  unroll=4)
        return tot

    total = lax.fori_loop(0, CH, chunk_body, jnp.zeros((L,), jnp.float32))
    obuf[...] = total
    pltpu.sync_copy(obuf, out_hbm.at[wid])


CBLK = 512
CT = K_TC // CBLK


def _tc_implies_body(p_ref, ia_ref, ib_ref, out_ref, a_oh, b_oh):
    j = pl.program_id(0)
    i = pl.program_id(1)

    @pl.when(i == 0)
    def _():
        rows = lax.broadcasted_iota(jnp.int32, (N, CBLK), 0)
        a_oh[...] = (rows == ia_ref[0, 0, :][None, :]).astype(jnp.bfloat16)
        b_oh[...] = (rows == ib_ref[0, 0, :][None, :]).astype(jnp.bfloat16)

    pb = p_ref[...].astype(jnp.bfloat16)
    pa = lax.dot_general(pb, a_oh[...], (((1,), (0,)), ((), ())),
                         preferred_element_type=jnp.float32)
    pc = lax.dot_general(pb, b_oh[...], (((1,), (0,)), ((), ())),
                         preferred_element_type=jnp.float32)
    s = jnp.sum(jnp.maximum(pa - pc, 0.0)).reshape(1, 1)

    @pl.when(jnp.logical_and(j == 0, i == 0))
    def _():
        out_ref[...] = s

    @pl.when(jnp.logical_or(j > 0, i > 0))
    def _():
        out_ref[...] += s


def _tc_implies_call():
    blk = 512
    return pl.pallas_call(
        _tc_implies_body,
        grid=(CT, B // blk),
        in_specs=[
            pl.BlockSpec((blk, N), lambda j, i: (i, 0)),
            pl.BlockSpec((1, 1, CBLK), lambda j, i: (j, 0, 0)),
            pl.BlockSpec((1, 1, CBLK), lambda j, i: (j, 0, 0)),
        ],
        out_specs=pl.BlockSpec((1, 1), lambda j, i: (0, 0)),
        out_shape=jax.ShapeDtypeStruct((1, 1), jnp.float32),
        scratch_shapes=[
            pltpu.VMEM((N, CBLK), jnp.bfloat16),
            pltpu.VMEM((N, CBLK), jnp.bfloat16),
        ],
    )


_tc_implies = _tc_implies_call()


BLK = 512
NBLK = B // BLK


def _tc_mutex_body(p_ref, ma_ref, mb_ref, out_ref, g_acc, m_acc):
    i = pl.program_id(0)
    pb = p_ref[...].astype(jnp.bfloat16)
    g_part = lax.dot_general(pb, pb, (((0,), (0,)), ((), ())),
                             preferred_element_type=jnp.float32)
    am = ma_ref[0, 0, :]
    bm = mb_ref[0, 0, :]
    cols = lax.broadcasted_iota(jnp.int32, (BLK, N), 1)
    a_oh = (cols == am[:, None]).astype(jnp.bfloat16)
    b_oh = (cols == bm[:, None]).astype(jnp.bfloat16)
    m_part = lax.dot_general(a_oh, b_oh, (((0,), (0,)), ((), ())),
                             preferred_element_type=jnp.float32)

    @pl.when(i == 0)
    def _():
        g_acc[...] = g_part
        m_acc[...] = m_part

    @pl.when(i > 0)
    def _():
        g_acc[...] += g_part
        m_acc[...] += m_part

    @pl.when(i == NBLK - 1)
    def _():
        out_ref[...] = jnp.sum(g_acc[...] * m_acc[...]).reshape(1, 1)


_tc_mutex = pl.pallas_call(
    _tc_mutex_body,
    grid=(NBLK,),
    in_specs=[
        pl.BlockSpec((BLK, N), lambda i: (i, 0)),
        pl.BlockSpec((1, 1, BLK), lambda i: (i, 0, 0)),
        pl.BlockSpec((1, 1, BLK), lambda i: (i, 0, 0)),
    ],
    out_specs=pl.BlockSpec((1, 1), lambda i: (0, 0)),
    out_shape=jax.ShapeDtypeStruct((1, 1), jnp.float32),
    scratch_shapes=[
        pltpu.VMEM((N, N), jnp.float32),
        pltpu.VMEM((N, N), jnp.float32),
    ],
)


def kernel(predictions, mutex_pairs, implies_pairs):
    ma = mutex_pairs[:, 0].astype(jnp.int32).reshape(NBLK, 1, BLK)
    mb = mutex_pairs[:, 1].astype(jnp.int32).reshape(NBLK, 1, BLK)
    ia = implies_pairs[:, 0].astype(jnp.int32)
    ib = implies_pairs[:, 1].astype(jnp.int32)
    ia_tc = ia[:K_TC].reshape(CT, 1, CBLK)
    ib_tc = ib[:K_TC].reshape(CT, 1, CBLK)
    partials = _sc_implies(predictions, ia[K_TC:], ib[K_TC:])
    mutex_sum = _tc_mutex(predictions, ma, mb)[0, 0]
    imp_tc = _tc_implies(predictions, ia_tc, ib_tc)[0, 0]
    return (jnp.sum(partials) + mutex_sum + imp_tc) * (1.0 / B)


# speedup vs baseline: 1.1051x; 1.0255x over previous
import functools

import numpy as np

import jax
import jax.numpy as jnp
from jax import lax
from jax.experimental import pallas as pl
from jax.experimental.pallas import tpu as pltpu
from jax.experimental.pallas import tpu_sc as plsc

B, N = 4096, 1000
NC, NS, L = 2, 16, 16
NW = NC * NS
ROWS_PER_W = B // NW
HALF = 64
K = 4096
K_TC = 1024
K_SC = K - K_TC
CH = K_SC // L

_mesh = plsc.VectorSubcoreMesh(core_axis_name="c", subcore_axis_name="s")


RP2 = ROWS_PER_W // 2
_HI_MASK = np.int32(-65536)


QROWS = 32
NQ = ROWS_PER_W // QROWS


@functools.partial(
    pl.kernel,
    out_type=jax.ShapeDtypeStruct((NW, L), jnp.float32),
    mesh=_mesh,
    scratch_types=[
        pltpu.VMEM((N * RP2,), jnp.int32),
        pltpu.VMEM((QROWS, N), jnp.float32),
        pltpu.VMEM((K_SC,), jnp.int32),
        pltpu.VMEM((K_SC,), jnp.int32),
        pltpu.VMEM((L,), jnp.float32),
    ],
    compiler_params=pltpu.CompilerParams(
        use_tc_tiling_on_sc=True, needs_layout_passes=False),
)
def _sc_implies(p_hbm, ia_hbm, ib_hbm, out_hbm, slab, fq, ia, ib, obuf):
    wid = lax.axis_index("s") * NC + lax.axis_index("c")
    pltpu.sync_copy(ia_hbm, ia)
    pltpu.sync_copy(ib_hbm, ib)

    def pack16(re, o0):
        def col_body(j, _):
            c = j * L
            a = fq[re, pl.ds(c, L)]
            b = fq[re + 1, pl.ds(c, L)]
            w = plsc.pack(a, b, format=plsc.PackFormat.INTERLEAVED)
            slab[pl.ds(o0 + c, L)] = plsc.bitcast(w, jnp.int32)
            return 0
        lax.fori_loop(0, N // L, col_body, 0, unroll=8)
        c = N - L
        a = fq[re, pl.ds(c, L)]
        b = fq[re + 1, pl.ds(c, L)]
        w = plsc.pack(a, b, format=plsc.PackFormat.INTERLEAVED)
        slab[pl.ds(o0 + c, L)] = plsc.bitcast(w, jnp.int32)

    for q in range(NQ):
        pltpu.sync_copy(p_hbm.at[wid, pl.ds(q * QROWS, QROWS), :], fq)

        def rp_body(rp, _):
            pack16(2 * rp, (q * (QROWS // 2) + rp) * N)
            return 0
        lax.fori_loop(0, QROWS // 2, rp_body, 0)

    def chunk_body(c, tot):
        av = ia[pl.ds(c * L, L)]
        bv = ib[pl.ds(c * L, L)]

        def group_body(g, carry):
            ja, jb, t = carry
            acc = jnp.zeros((2 * L,), jnp.bfloat16)
            for _ in range(4):
                wa = plsc.load_gather(slab, [ja])
                wb = plsc.load_gather(slab, [jb])
                da = plsc.bitcast(wa, jnp.bfloat16)
                db = plsc.bitcast(wb, jnp.bfloat16)
                acc = acc + jnp.maximum(da - db, 0)
                ja = ja + N
                jb = jb + N
            ai = plsc.bitcast(acc, jnp.int32)
            t = t + plsc.bitcast(lax.shift_left(ai, 16), jnp.float32)
            t = t + plsc.bitcast(lax.bitwise_and(ai, _HI_MASK), jnp.float32)
            return ja, jb, t

        _, _, tot = lax.fori_loop(0, RP2 // 4, group_body, (av, bv, tot),
                                  unroll=4)
        return tot

    total = lax.fori_loop(0, CH, chunk_body, jnp.zeros((L,), jnp.float32))
    obuf[...] = total
    pltpu.sync_copy(obuf, out_hbm.at[wid])


CBLK = 512
CT = K_TC // CBLK


def _tc_implies_body(p_ref, ia_ref, ib_ref, out_ref, a_oh, b_oh):
    j = pl.program_id(0)
    i = pl.program_id(1)

    @pl.when(i == 0)
    def _():
        rows = lax.broadcasted_iota(jnp.int32, (N, CBLK), 0)
        a_oh[...] = (rows == ia_ref[0, 0, :][None, :]).astype(jnp.bfloat16)
        b_oh[...] = (rows == ib_ref[0, 0, :][None, :]).astype(jnp.bfloat16)

    pb = p_ref[...].astype(jnp.bfloat16)
    pa = lax.dot_general(pb, a_oh[...], (((1,), (0,)), ((), ())),
                         preferred_element_type=jnp.float32)
    pc = lax.dot_general(pb, b_oh[...], (((1,), (0,)), ((), ())),
                         preferred_element_type=jnp.float32)
    s = jnp.sum(jnp.maximum(pa - pc, 0.0)).reshape(1, 1)

    @pl.when(jnp.logical_and(j == 0, i == 0))
    def _():
        out_ref[...] = s

    @pl.when(jnp.logical_or(j > 0, i > 0))
    def _():
        out_ref[...] += s


def _tc_implies_call():
    blk = 512
    return pl.pallas_call(
        _tc_implies_body,
        grid=(CT, B // blk),
        in_specs=[
            pl.BlockSpec((blk, N), lambda j, i: (i, 0)),
            pl.BlockSpec((1, 1, CBLK), lambda j, i: (j, 0, 0)),
            pl.BlockSpec((1, 1, CBLK), lambda j, i: (j, 0, 0)),
        ],
        out_specs=pl.BlockSpec((1, 1), lambda j, i: (0, 0)),
        out_shape=jax.ShapeDtypeStruct((1, 1), jnp.float32),
        scratch_shapes=[
            pltpu.VMEM((N, CBLK), jnp.bfloat16),
            pltpu.VMEM((N, CBLK), jnp.bfloat16),
        ],
    )


_tc_implies = _tc_implies_call()


BLK = 512
NBLK = B // BLK


def _tc_mutex_body(p_ref, ma_ref, mb_ref, out_ref, g_acc, m_acc):
    i = pl.program_id(0)
    pb = p_ref[...].astype(jnp.bfloat16)
    g_part = lax.dot_general(pb, pb, (((0,), (0,)), ((), ())),
                             preferred_element_type=jnp.float32)
    am = ma_ref[0, 0, :]
    bm = mb_ref[0, 0, :]
    cols = lax.broadcasted_iota(jnp.int32, (BLK, N), 1)
    a_oh = (cols == am[:, None]).astype(jnp.bfloat16)
    b_oh = (cols == bm[:, None]).astype(jnp.bfloat16)
    m_part = lax.dot_general(a_oh, b_oh, (((0,), (0,)), ((), ())),
                             preferred_element_type=jnp.float32)

    @pl.when(i == 0)
    def _():
        g_acc[...] = g_part
        m_acc[...] = m_part

    @pl.when(i > 0)
    def _():
        g_acc[...] += g_part
        m_acc[...] += m_part

    @pl.when(i == NBLK - 1)
    def _():
        out_ref[...] = jnp.sum(g_acc[...] * m_acc[...]).reshape(1, 1)


_tc_mutex = pl.pallas_call(
    _tc_mutex_body,
    grid=(NBLK,),
    in_specs=[
        pl.BlockSpec((BLK, N), lambda i: (i, 0)),
        pl.BlockSpec((1, 1, BLK), lambda i: (i, 0, 0)),
        pl.BlockSpec((1, 1, BLK), lambda i: (i, 0, 0)),
    ],
    out_specs=pl.BlockSpec((1, 1), lambda i: (0, 0)),
    out_shape=jax.ShapeDtypeStruct((1, 1), jnp.float32),
    scratch_shapes=[
        pltpu.VMEM((N, N), jnp.float32),
        pltpu.VMEM((N, N), jnp.float32),
    ],
)


def kernel(predictions, mutex_pairs, implies_pairs):
    ma = mutex_pairs[:, 0].astype(jnp.int32).reshape(NBLK, 1, BLK)
    mb = mutex_pairs[:, 1].astype(jnp.int32).reshape(NBLK, 1, BLK)
    ia = implies_pairs[:, 0].astype(jnp.int32)
    ib = implies_pairs[:, 1].astype(jnp.int32)
    ia_tc = ia[:K_TC].reshape(CT, 1, CBLK)
    ib_tc = ib[:K_TC].reshape(CT, 1, CBLK)
    partials = _sc_implies(predictions.reshape(NW, ROWS_PER_W, N),
                           ia[K_TC:], ib[K_TC:])
    mutex_sum = _tc_mutex(predictions, ma, mb)[0, 0]
    imp_tc = _tc_implies(predictions, ia_tc, ib_tc)[0, 0]
    return (jnp.sum(partials) + mutex_sum + imp_tc) * (1.0 / B)
